# Initial kernel scaffold; baseline (speedup 1.0000x reference)
import jax
import jax.numpy as jnp
from jax.experimental import pallas as pl


def kernel(x, edge_index, Wl1, bl1, Wr1, Wl2, bl2, Wr2):
    raise NotImplementedError("write your pallas kernel here")


# speedup vs baseline: 4.4792x; 4.4792x over previous
import functools

import jax
import jax.numpy as jnp
from jax import lax
from jax.experimental import pallas as pl
from jax.experimental.pallas import tpu as pltpu
from jax.experimental.pallas import tpu_sc as plsc

N_NODES = 10000
N_EDGES = 320000
D = 128

NC = 2
NS = 16
NW = NC * NS
K = 128
G = 79
C = G * K
E_PAD = NW * C
TRASH = 10016
NROW = 10240
RPT = NROW // NS


def _spmm_body(with_cnt, x_hbm, src_hbm, dst_hbm, *rest):
    if with_cnt:
        (agg_out, cnt_out, idx_v, dst_v, rows_v, ones_v, zbuf, acc_sh, cnt_sh, sem) = rest
    else:
        (agg_out, idx_v, dst_v, rows_v, zbuf, acc_sh, sem) = rest

    c = lax.axis_index("c")
    s = lax.axis_index("s")

    zero16 = jnp.zeros((16,), jnp.float32)
    ones16 = jnp.ones((16,), jnp.float32)

    for i in range(16):
        for j in range(8):
            zbuf[i, pl.ds(j * 16, 16)] = zero16

    tb = s * RPT

    @pl.loop(0, RPT // 16)
    def _zero_acc(k):
        pltpu.sync_copy(zbuf, acc_sh.at[pl.ds(tb + k * 16, 16), :])

    if with_cnt:
        for j in range(K // 16):
            ones_v[pl.ds(j * 16, 16)] = ones16
        pltpu.sync_copy(zbuf.at[0], cnt_sh.at[pl.ds(tb, D)])
        pltpu.sync_copy(zbuf.at[0], cnt_sh.at[pl.ds(tb + D, D)])
        pltpu.sync_copy(zbuf.at[0], cnt_sh.at[pl.ds(tb + 2 * D, D)])
        pltpu.sync_copy(zbuf.at[0], cnt_sh.at[pl.ds(tb + 3 * D, D)])
        pltpu.sync_copy(zbuf.at[0], cnt_sh.at[pl.ds(tb + 4 * D, D)])

    plsc.subcore_barrier()

    wid = s * NC + c
    ebase = wid * C

    @pl.loop(0, G)
    def _edges(g):
        base = ebase + g * K
        pltpu.sync_copy(src_hbm.at[pl.ds(base, K)], idx_v)
        pltpu.sync_copy(dst_hbm.at[pl.ds(base, K)], dst_v)
        pltpu.async_copy(x_hbm.at[idx_v], rows_v, sem).wait()
        pltpu.sync_copy(rows_v, acc_sh.at[dst_v], add=True)
        if with_cnt:
            pltpu.sync_copy(ones_v, cnt_sh.at[dst_v], add=True)

    plsc.subcore_barrier()

    pltpu.sync_copy(acc_sh.at[pl.ds(tb, RPT), :], agg_out.at[c, pl.ds(tb, RPT), :])
    if with_cnt:
        pltpu.sync_copy(cnt_sh.at[pl.ds(tb, RPT)], cnt_out.at[c, pl.ds(tb, RPT)])


def _make_spmm(with_cnt):
    mesh = plsc.VectorSubcoreMesh(core_axis_name="c", subcore_axis_name="s")
    out_type = [jax.ShapeDtypeStruct((NC, NROW, D), jnp.float32)]
    scratch = [
        pltpu.VMEM((K,), jnp.int32),
        pltpu.VMEM((K,), jnp.int32),
        pltpu.VMEM((K, D), jnp.float32),
    ]
    if with_cnt:
        out_type.append(jax.ShapeDtypeStruct((NC, NROW), jnp.float32))
        scratch.append(pltpu.VMEM((K,), jnp.float32))
    scratch.append(pltpu.VMEM((16, D), jnp.float32))
    scratch.append(pltpu.VMEM_SHARED((NROW, D), jnp.float32))
    if with_cnt:
        scratch.append(pltpu.VMEM_SHARED((NROW,), jnp.float32))
    scratch.append(pltpu.SemaphoreType.DMA)
    return pl.kernel(
        functools.partial(_spmm_body, with_cnt),
        out_type=out_type,
        mesh=mesh,
        scratch_types=scratch,
        name="sage_spmm_cnt" if with_cnt else "sage_spmm",
    )


_spmm1 = _make_spmm(True)
_spmm2 = _make_spmm(False)

_BLK = 1024


def _epi_body(relu, agg_ref, cnt_ref, x_ref, wl_ref, bl_ref, wr_ref, o_ref):
    agg = agg_ref[0] + agg_ref[1]
    cnt = jnp.sum(cnt_ref[...], axis=0)
    inv = 1.0 / jnp.maximum(cnt, 1.0)
    mean = agg * inv[:, None]
    h = (jnp.dot(mean, wl_ref[...], preferred_element_type=jnp.float32)
         + bl_ref[...]
         + jnp.dot(x_ref[...], wr_ref[...], preferred_element_type=jnp.float32))
    if relu:
        h = jnp.where(h >= 0, h, 0.01 * h)
    o_ref[...] = h


def _make_epi(relu):
    return pl.pallas_call(
        functools.partial(_epi_body, relu),
        grid=(NROW // _BLK,),
        in_specs=[
            pl.BlockSpec((NC, _BLK, D), lambda i: (0, i, 0)),
            pl.BlockSpec((NC, _BLK), lambda i: (0, i)),
            pl.BlockSpec((_BLK, D), lambda i: (i, 0)),
            pl.BlockSpec((D, D), lambda i: (0, 0)),
            pl.BlockSpec((1, D), lambda i: (0, 0)),
            pl.BlockSpec((D, D), lambda i: (0, 0)),
        ],
        out_specs=pl.BlockSpec((_BLK, D), lambda i: (i, 0)),
        out_shape=jax.ShapeDtypeStruct((NROW, D), jnp.float32),
    )


_epi1 = _make_epi(True)
_epi2 = _make_epi(False)


def kernel(x, edge_index, Wl1, bl1, Wr1, Wl2, bl2, Wr2):
    src = edge_index[0].astype(jnp.int32)
    dst = edge_index[1].astype(jnp.int32)
    pad = E_PAD - N_EDGES
    src_p = jnp.concatenate([src, jnp.zeros((pad,), jnp.int32)])
    dst_p = jnp.concatenate([dst, jnp.full((pad,), TRASH, jnp.int32)])
    x_p = jnp.pad(x, ((0, NROW - N_NODES), (0, 0)))

    agg1, cntp = _spmm1(x_p, src_p, dst_p)
    h = _epi1(agg1, cntp, x_p, Wl1.T, bl1.reshape(1, D), Wr1.T)
    (agg2,) = _spmm2(h, src_p, dst_p)
    out = _epi2(agg2, cntp, h, Wl2.T, bl2.reshape(1, D), Wr2.T)
    return out[:N_NODES]
